# baseline (speedup 1.0000x reference)
import functools

import jax
import jax.numpy as jnp
from jax import lax
from jax.experimental import pallas as pl
from jax.experimental.pallas import tpu as pltpu
from jax.experimental.pallas import tpu_sc as plsc

BATCH = 32
SEQ_LEN = 512
PRED_LEN = 336
ENC_IN = 16
D_MODEL = 1024
NUM_EXPERTS = 8
BN = BATCH * ENC_IN
TILE = 64
RPAD = 1536
NT = RPAD // TILE
NASSIGN = 2 * BN

_EXACT = lax.Precision.HIGHEST


def _router_body(xt_ref, wg_ref, ci_ref, sort_src_ref, comb_ref, tokp_ref,
                 te_ref, tv_ref):
    x = xt_ref[...]
    m = jnp.mean(x, axis=1, keepdims=True)
    xc = x - m
    var = jnp.mean(xc * xc, axis=1, keepdims=True)
    std = jnp.sqrt(var + 1e-5)
    ci = xc / std
    ci_ref[...] = ci

    logits = jnp.dot(ci, wg_ref[...], preferred_element_type=jnp.float32)
    io8 = lax.broadcasted_iota(jnp.int32, (BN, NUM_EXPERTS), 1)
    v1 = jnp.max(logits, axis=1, keepdims=True)
    e1 = jnp.min(jnp.where(logits == v1, io8, NUM_EXPERTS), axis=1,
                 keepdims=True)
    l2 = jnp.where(io8 == e1, -1e30, logits)
    v2 = jnp.max(l2, axis=1, keepdims=True)
    e2 = jnp.min(jnp.where(l2 == v2, io8, NUM_EXPERTS), axis=1, keepdims=True)
    g1 = 1.0 / (1.0 + jnp.exp(v2 - v1))
    g2 = 1.0 - g1

    h0 = (io8 == e1).astype(jnp.float32)
    h1 = (io8 == e2).astype(jnp.float32)

    r_io = lax.broadcasted_iota(jnp.int32, (BN, BN), 0)
    c_io = lax.broadcasted_iota(jnp.int32, (BN, BN), 1)
    tstrict = (c_io < r_io).astype(jnp.float32)
    ecs0 = jnp.dot(tstrict, h0, precision=_EXACT,
                   preferred_element_type=jnp.float32)
    ecs1 = jnp.dot(tstrict, h1, precision=_EXACT,
                   preferred_element_type=jnp.float32)
    c0 = jnp.sum(h0, axis=0, keepdims=True)
    cnt = c0 + jnp.sum(h1, axis=0, keepdims=True)
    cnt_i = cnt.astype(jnp.int32)
    pc = ((cnt_i + (TILE - 1)) // TILE) * TILE
    pc_f = pc.astype(jnp.float32)

    e_r = lax.broadcasted_iota(jnp.int32, (NUM_EXPERTS, NUM_EXPERTS), 0)
    e_c = lax.broadcasted_iota(jnp.int32, (NUM_EXPERTS, NUM_EXPERTS), 1)
    pc_col = jnp.sum(pc_f * (e_r == e_c), axis=1, keepdims=True)
    po = jnp.sum(pc_col * (e_r < e_c), axis=0, keepdims=True)

    rank0 = jnp.sum(ecs0 * h0, axis=1, keepdims=True)
    rank1 = (jnp.sum(ecs1 * h1, axis=1, keepdims=True)
             + jnp.sum(c0 * h1, axis=1, keepdims=True))
    d0 = jnp.sum(po * h0, axis=1, keepdims=True) + rank0
    d1 = jnp.sum(po * h1, axis=1, keepdims=True) + rank1
    comb_ref[...] = jnp.concatenate([d0, d1], axis=0).astype(jnp.int32)

    s_io = lax.broadcasted_iota(jnp.int32, (BN, RPAD), 1).astype(jnp.float32)
    tok = lax.broadcasted_iota(jnp.int32, (BN, RPAD), 0).astype(jnp.float32)
    contrib = (jnp.where(d0 == s_io, tok, 0.0)
               + jnp.where(d1 == s_io, tok, 0.0))
    sort_src_ref[...] = jnp.sum(contrib, axis=0,
                                keepdims=True).astype(jnp.int32)

    io4 = lax.broadcasted_iota(jnp.int32, (BN, 4), 1)
    tokp_ref[...] = (g1 * (io4 == 0) + g2 * (io4 == 1)
                     + std * (io4 == 2) + m * (io4 == 3))

    j64 = (lax.broadcasted_iota(jnp.int32, (NUM_EXPERTS, NT), 1)
           * TILE).astype(jnp.float32)
    po_col = jnp.sum(po * (e_r == e_c), axis=1, keepdims=True)
    te = jnp.sum((po_col <= j64).astype(jnp.float32), axis=0,
                 keepdims=True) - 1.0
    te_ref[...] = te.astype(jnp.int32)
    total_pad = jnp.sum(pc_f)
    j1 = (lax.broadcasted_iota(jnp.int32, (1, NT), 1)
          * TILE).astype(jnp.float32)
    tv_ref[...] = (j1 < total_pad).astype(jnp.int32)


def _group_body(te_ref, tv_ref, x_ref, w_ref, b_ref, h_ref):
    j = pl.program_id(0)

    @pl.when(tv_ref[j] == 1)
    def _():
        h_ref[...] = jnp.maximum(
            jnp.dot(x_ref[...], w_ref[0], preferred_element_type=jnp.float32)
            + b_ref[0], 0.0)


def _head_body(y2_ref, tokp_ref, rv_ref, hw_ref, hb_ref, out_ref):
    y2 = y2_ref[...]
    ya = y2[:BN]
    yb = y2[BN:]
    g1 = tokp_ref[:, 0:1]
    g2 = tokp_ref[:, 1:2]
    std = tokp_ref[:, 2:3]
    mn = tokp_ref[:, 3:4]
    y = g1 * ya + g2 * yb
    z = jnp.dot(y, hw_ref[...], preferred_element_type=jnp.float32) \
        + hb_ref[...]
    out_ref[...] = (z * rv_ref[:, 0:1] + rv_ref[:, 1:2]) * std + mn


def _sc_gather(table, idx, n_rows, d):
    mesh = plsc.VectorSubcoreMesh(core_axis_name="c", subcore_axis_name="s")
    nw = mesh.num_cores * mesh.num_subcores
    b_per_w = n_rows // nw

    @functools.partial(
        pl.kernel,
        out_type=jax.ShapeDtypeStruct((n_rows, d), jnp.float32),
        mesh=mesh,
        scratch_types=[
            pltpu.VMEM((b_per_w,), jnp.int32),
            pltpu.VMEM((b_per_w, d), jnp.float32),
            pltpu.SemaphoreType.DMA,
        ],
    )
    def k(table_hbm, idx_hbm, out_hbm, idx_v, rows_v, sem):
        wid = lax.axis_index("s") * mesh.num_cores + lax.axis_index("c")
        base = wid * b_per_w
        pltpu.sync_copy(idx_hbm.at[pl.ds(base, b_per_w)], idx_v)
        pltpu.async_copy(table_hbm.at[idx_v], rows_v, sem).wait()
        pltpu.sync_copy(rows_v, out_hbm.at[pl.ds(base, b_per_w)])

    return k(table, idx)


@jax.jit
def kernel(x_enc, x_mark_enc, x_dec, x_mark_dec, w_gate, expert_W, expert_b,
           head_W, head_b, revin_w, revin_b):
    xt = jnp.transpose(x_enc, (0, 2, 1)).reshape(BN, SEQ_LEN)
    rv = jnp.stack([jnp.tile(revin_w, BATCH), jnp.tile(revin_b, BATCH)],
                   axis=1)

    ci, sort_src, comb, tokp, te, tv = pl.pallas_call(
        _router_body,
        out_shape=[
            jax.ShapeDtypeStruct((BN, SEQ_LEN), jnp.float32),
            jax.ShapeDtypeStruct((1, RPAD), jnp.int32),
            jax.ShapeDtypeStruct((NASSIGN, 1), jnp.int32),
            jax.ShapeDtypeStruct((BN, 4), jnp.float32),
            jax.ShapeDtypeStruct((1, NT), jnp.int32),
            jax.ShapeDtypeStruct((1, NT), jnp.int32),
        ],
    )(xt, w_gate)

    x_sorted = _sc_gather(ci, sort_src.reshape(RPAD), RPAD, SEQ_LEN)

    h_sorted = pl.pallas_call(
        _group_body,
        grid_spec=pltpu.PrefetchScalarGridSpec(
            num_scalar_prefetch=2,
            grid=(NT,),
            in_specs=[
                pl.BlockSpec((TILE, SEQ_LEN), lambda j, te, tv: (j, 0)),
                pl.BlockSpec((1, SEQ_LEN, D_MODEL),
                             lambda j, te, tv: (te[j], 0, 0)),
                pl.BlockSpec((1, 1, D_MODEL), lambda j, te, tv: (te[j], 0, 0)),
            ],
            out_specs=pl.BlockSpec((TILE, D_MODEL), lambda j, te, tv: (j, 0)),
        ),
        out_shape=jax.ShapeDtypeStruct((RPAD, D_MODEL), jnp.float32),
        compiler_params=pltpu.CompilerParams(
            dimension_semantics=("arbitrary",)),
    )(te.reshape(NT), tv.reshape(NT), x_sorted, expert_W,
      expert_b.reshape(NUM_EXPERTS, 1, D_MODEL))

    y2 = _sc_gather(h_sorted, comb.reshape(NASSIGN), NASSIGN, D_MODEL)

    out_tok = pl.pallas_call(
        _head_body,
        out_shape=jax.ShapeDtypeStruct((BN, PRED_LEN), jnp.float32),
    )(y2, tokp, rv, head_W, head_b.reshape(1, PRED_LEN))

    return out_tok.reshape(BATCH, ENC_IN, PRED_LEN).transpose(0, 2, 1)


# speedup vs baseline: 3.3178x; 3.3178x over previous
import functools

import jax
import jax.numpy as jnp
from jax.experimental import pallas as pl
from jax.experimental.pallas import tpu as pltpu

BATCH = 32
SEQ_LEN = 512
PRED_LEN = 336
ENC_IN = 16
D_MODEL = 1024
NUM_EXPERTS = 8
BN = BATCH * ENC_IN


def _fused_body(xt_ref, wg_ref, ew_ref, eb_ref, hw_ref, hb_ref, rv_ref,
                out_ref, y_acc):
    e = pl.program_id(0)
    x = xt_ref[...]
    m = jnp.mean(x, axis=1, keepdims=True)
    xc = x - m
    var = jnp.mean(xc * xc, axis=1, keepdims=True)
    std = jnp.sqrt(var + 1e-5)
    ci = xc / std

    logits = jnp.dot(ci, wg_ref[...], preferred_element_type=jnp.float32)
    io = jax.lax.broadcasted_iota(jnp.int32, (BN, NUM_EXPERTS), 1)
    v1 = jnp.max(logits, axis=1, keepdims=True)
    e1 = jnp.min(jnp.where(logits == v1, io, NUM_EXPERTS), axis=1,
                 keepdims=True)
    l2 = jnp.where(io == e1, -1e30, logits)
    v2 = jnp.max(l2, axis=1, keepdims=True)
    e2 = jnp.min(jnp.where(l2 == v2, io, NUM_EXPERTS), axis=1, keepdims=True)
    g1 = 1.0 / (1.0 + jnp.exp(v2 - v1))
    g2 = 1.0 - g1
    gate_e = g1 * (e1 == e) + g2 * (e2 == e)

    eo = jnp.maximum(
        jnp.dot(ci.astype(jnp.bfloat16), ew_ref[0].astype(jnp.bfloat16),
                preferred_element_type=jnp.float32)
        + eb_ref[0], 0.0)

    @pl.when(e == 0)
    def _():
        y_acc[...] = gate_e * eo

    @pl.when(e > 0)
    def _():
        y_acc[...] += gate_e * eo

    @pl.when(e == NUM_EXPERTS - 1)
    def _():
        z = jnp.dot(y_acc[...].astype(jnp.bfloat16),
                    hw_ref[...].astype(jnp.bfloat16),
                    preferred_element_type=jnp.float32) + hb_ref[...]
        rw = rv_ref[:, 0:1]
        rb = rv_ref[:, 1:2]
        out_ref[...] = (z * rw + rb) * std + m


@jax.jit
def kernel(x_enc, x_mark_enc, x_dec, x_mark_dec, w_gate, expert_W, expert_b,
           head_W, head_b, revin_w, revin_b):
    xt = jnp.transpose(x_enc, (0, 2, 1)).reshape(BN, SEQ_LEN)
    rv = jnp.stack([jnp.tile(revin_w, BATCH), jnp.tile(revin_b, BATCH)],
                   axis=1)

    out_tok = pl.pallas_call(
        _fused_body,
        grid=(NUM_EXPERTS,),
        in_specs=[
            pl.BlockSpec((BN, SEQ_LEN), lambda e: (0, 0)),
            pl.BlockSpec((SEQ_LEN, NUM_EXPERTS), lambda e: (0, 0)),
            pl.BlockSpec((1, SEQ_LEN, D_MODEL), lambda e: (e, 0, 0)),
            pl.BlockSpec((1, 1, D_MODEL), lambda e: (e, 0, 0)),
            pl.BlockSpec((D_MODEL, PRED_LEN), lambda e: (0, 0)),
            pl.BlockSpec((1, PRED_LEN), lambda e: (0, 0)),
            pl.BlockSpec((BN, 2), lambda e: (0, 0)),
        ],
        out_specs=pl.BlockSpec((BN, PRED_LEN), lambda e: (0, 0)),
        out_shape=jax.ShapeDtypeStruct((BN, PRED_LEN), jnp.float32),
        scratch_shapes=[pltpu.VMEM((BN, D_MODEL), jnp.float32)],
        compiler_params=pltpu.CompilerParams(
            dimension_semantics=("arbitrary",)),
    )(xt, w_gate, expert_W, expert_b.reshape(NUM_EXPERTS, 1, D_MODEL),
      head_W, head_b.reshape(1, PRED_LEN), rv)

    return out_tok.reshape(BATCH, ENC_IN, PRED_LEN).transpose(0, 2, 1)


# speedup vs baseline: 3.4223x; 1.0315x over previous
import jax
import jax.numpy as jnp
from jax import lax
from jax.experimental import pallas as pl
from jax.experimental.pallas import tpu as pltpu

BATCH = 32
SEQ_LEN = 512
PRED_LEN = 336
ENC_IN = 16
D_MODEL = 1024
NUM_EXPERTS = 8
BN = BATCH * ENC_IN


def _fused_body(xt_ref, wg_ref, ew_ref, eb_ref, hw_ref, hb_ref, rv_ref,
                out_ref, ci_bf, gall, stm, y_acc):
    e = pl.program_id(0)

    @pl.when(e == 0)
    def _():
        x = xt_ref[...]
        m = jnp.mean(x, axis=1, keepdims=True)
        xc = x - m
        var = jnp.mean(xc * xc, axis=1, keepdims=True)
        std = jnp.sqrt(var + 1e-5)
        ci = xc / std
        ci_bf[...] = ci.astype(jnp.bfloat16)
        stm[...] = jnp.concatenate([std, m], axis=1)

        logits = jnp.dot(ci, wg_ref[...], preferred_element_type=jnp.float32)
        io = lax.broadcasted_iota(jnp.int32, (BN, NUM_EXPERTS), 1)
        v1 = jnp.max(logits, axis=1, keepdims=True)
        e1 = jnp.min(jnp.where(logits == v1, io, NUM_EXPERTS), axis=1,
                     keepdims=True)
        l2 = jnp.where(io == e1, -1e30, logits)
        v2 = jnp.max(l2, axis=1, keepdims=True)
        e2 = jnp.min(jnp.where(l2 == v2, io, NUM_EXPERTS), axis=1,
                     keepdims=True)
        g1 = 1.0 / (1.0 + jnp.exp(v2 - v1))
        g2 = 1.0 - g1
        gall[...] = g1 * (io == e1) + g2 * (io == e2)

    io8 = lax.broadcasted_iota(jnp.int32, (BN, NUM_EXPERTS), 1)
    gate_e = jnp.sum(gall[...] * (io8 == e), axis=1, keepdims=True)

    eo = jnp.maximum(
        jnp.dot(ci_bf[...], ew_ref[0].astype(jnp.bfloat16),
                preferred_element_type=jnp.float32)
        + eb_ref[0], 0.0)

    @pl.when(e == 0)
    def _():
        y_acc[...] = gate_e * eo

    @pl.when(e > 0)
    def _():
        y_acc[...] += gate_e * eo

    @pl.when(e == NUM_EXPERTS - 1)
    def _():
        z = jnp.dot(y_acc[...].astype(jnp.bfloat16),
                    hw_ref[...].astype(jnp.bfloat16),
                    preferred_element_type=jnp.float32) + hb_ref[...]
        rw = rv_ref[:, 0:1]
        rb = rv_ref[:, 1:2]
        std = stm[:, 0:1]
        m = stm[:, 1:2]
        out_ref[...] = (z * rw + rb) * std + m


@jax.jit
def kernel(x_enc, x_mark_enc, x_dec, x_mark_dec, w_gate, expert_W, expert_b,
           head_W, head_b, revin_w, revin_b):
    xt = jnp.transpose(x_enc, (0, 2, 1)).reshape(BN, SEQ_LEN)
    rv = jnp.stack([jnp.tile(revin_w, BATCH), jnp.tile(revin_b, BATCH)],
                   axis=1)

    out_tok = pl.pallas_call(
        _fused_body,
        grid=(NUM_EXPERTS,),
        in_specs=[
            pl.BlockSpec((BN, SEQ_LEN), lambda e: (0, 0)),
            pl.BlockSpec((SEQ_LEN, NUM_EXPERTS), lambda e: (0, 0)),
            pl.BlockSpec((1, SEQ_LEN, D_MODEL), lambda e: (e, 0, 0)),
            pl.BlockSpec((1, 1, D_MODEL), lambda e: (e, 0, 0)),
            pl.BlockSpec((D_MODEL, PRED_LEN), lambda e: (0, 0)),
            pl.BlockSpec((1, PRED_LEN), lambda e: (0, 0)),
            pl.BlockSpec((BN, 2), lambda e: (0, 0)),
        ],
        out_specs=pl.BlockSpec((BN, PRED_LEN), lambda e: (0, 0)),
        out_shape=jax.ShapeDtypeStruct((BN, PRED_LEN), jnp.float32),
        scratch_shapes=[
            pltpu.VMEM((BN, SEQ_LEN), jnp.bfloat16),
            pltpu.VMEM((BN, NUM_EXPERTS), jnp.float32),
            pltpu.VMEM((BN, 2), jnp.float32),
            pltpu.VMEM((BN, D_MODEL), jnp.float32),
        ],
        compiler_params=pltpu.CompilerParams(
            dimension_semantics=("arbitrary",)),
    )(xt, w_gate, expert_W, expert_b.reshape(NUM_EXPERTS, 1, D_MODEL),
      head_W, head_b.reshape(1, PRED_LEN), rv)

    return out_tok.reshape(BATCH, ENC_IN, PRED_LEN).transpose(0, 2, 1)
